# baseline (speedup 1.0000x reference)
import functools

import jax
import jax.numpy as jnp
from jax import lax
from jax.experimental import pallas as pl
from jax.experimental.pallas import tpu as pltpu
from jax.experimental.pallas import tpu_sc as plsc

_M = 1000000
_D = 64
_B = 16384
_NW = 32
_R = 31248
_EXTRA_BASE = _NW * _R
_LCAP = _B + 80
_CHUNK = 64
_CPB = 5000

_mesh = plsc.VectorSubcoreMesh(core_axis_name="c", subcore_axis_name="s")

_GDN = lax.GatherDimensionNumbers(
    offset_dims=(), collapsed_slice_dims=(0,), start_index_map=(0,))


def _permute(x, idx):
    return lax.gather(x, idx[:, None], dimension_numbers=_GDN,
                      slice_sizes=(1,),
                      mode=lax.GatherScatterMode.PROMISE_IN_BOUNDS)


@functools.partial(
    pl.kernel,
    out_type=(
        jax.ShapeDtypeStruct((_NW, _LCAP), jnp.int32),
        jax.ShapeDtypeStruct((_NW, _LCAP), jnp.int32),
        jax.ShapeDtypeStruct((_NW, 16), jnp.int32),
    ),
    mesh=_mesh,
    scratch_types=[
        pltpu.VMEM((_B,), jnp.int32),
        pltpu.VMEM((_LCAP,), jnp.int32),
        pltpu.VMEM((_LCAP,), jnp.int32),
        pltpu.VMEM((16,), jnp.int32),
    ],
    compiler_params=pltpu.CompilerParams(needs_layout_passes=False),
    cost_estimate=pl.CostEstimate(
        flops=2_000_000, bytes_accessed=600_000_000, transcendentals=0),
)
def _sc_prep(idx_hbm, rowl_hbm, wl_hbm, cnt_hbm,
             idx_v, rowlist_v, wlist_v, cnt_v):
    g = lax.axis_index("s") * 2 + lax.axis_index("c")
    lo = g * _R
    e_lo = _EXTRA_BASE + g * 2
    lane = lax.iota(jnp.int32, 16)

    pltpu.sync_copy(idx_hbm, idx_v)

    def _build(t, offv):
        v = idx_v[pl.ds(t * 16, 16)]
        ordv = lane + t * 16
        m = ((v >= lo) & (v < lo + _R)) | ((v >= e_lo) & (v < e_lo + 2))
        mi = m.astype(jnp.int32)
        pos = offv + plsc.cumsum(mi) - mi
        plsc.store_scatter(rowlist_v, [pos], v, mask=m)
        plsc.store_scatter(wlist_v, [pos], ordv, mask=m)
        return offv + plsc.all_reduce_population_count(m)
    offv = lax.fori_loop(0, _B // 16, _build, jnp.zeros((16,), jnp.int32))
    n = jnp.max(offv)

    cnt_v[pl.ds(0, 16)] = jnp.zeros((16,), jnp.int32) + n

    @pl.when(n > 0)
    def _dedup_and_pad():
        t_last = (n - 1) // 16 * 16
        lv_r = rowlist_v[pl.ds(t_last, 16)]
        lv_w = wlist_v[pl.ds(t_last, 16)]
        sel = (n - 1) - t_last
        r_last = jnp.max(jnp.where(lane == sel, lv_r, -1))
        w_last = jnp.max(jnp.where(lane == sel, lv_w, -1))
        rpad = jnp.zeros((16,), jnp.int32) + r_last
        wpad = jnp.zeros((16,), jnp.int32) + w_last
        t0 = n // 16

        @pl.loop(t0, t0 + 5)
        def _fill(t):
            cur_r = rowlist_v[pl.ds(t * 16, 16)]
            cur_w = wlist_v[pl.ds(t * 16, 16)]
            mm = (lane + t * 16) >= n
            rowlist_v[pl.ds(t * 16, 16)] = jnp.where(mm, rpad, cur_r)
            wlist_v[pl.ds(t * 16, 16)] = jnp.where(mm, wpad, cur_w)

        nchunks = (n + _CHUNK - 1) // _CHUNK
        nv = _CHUNK // 16

        @pl.loop(0, nchunks)
        def _dedup(c):
            base = c * _CHUNK
            cur = [rowlist_v[pl.ds(base + t * 16, 16)] for t in range(nv)]
            losers = []
            for t in range(nv):
                loser_t = lane < 0
                for u in range(t, nv):
                    ru = cur[u]
                    for k in range(16):
                        if t == u and k == 0:
                            continue
                        rot = _permute(ru, (lane + k) & 15) if k else ru
                        rotpos = ((lane + k) & 15) + (u - t) * 16
                        valid = rotpos > lane
                        loser_t = loser_t | ((rot == cur[t]) & valid)
                losers.append(loser_t)
            for t in range(nv):
                cw = wlist_v[pl.ds(base + t * 16, 16)]
                rowlist_v[pl.ds(base + t * 16, 16)] = (
                    jnp.where(losers[t], rpad, cur[t]))
                wlist_v[pl.ds(base + t * 16, 16)] = (
                    jnp.where(losers[t], wpad, cw))

    pltpu.sync_copy(rowlist_v, rowl_hbm.at[g])
    pltpu.sync_copy(wlist_v, wl_hbm.at[g])
    pltpu.sync_copy(cnt_v, cnt_hbm.at[g])


def _copy_body(x_ref, o_ref):
    o_ref[...] = x_ref[...]


_tc_copy = pl.pallas_call(
    _copy_body,
    out_shape=jax.ShapeDtypeStruct((_M, _D), jnp.float32),
    grid=(_M // _CPB,),
    in_specs=[pl.BlockSpec((_CPB, _D), lambda i: (i, 0))],
    out_specs=pl.BlockSpec((_CPB, _D), lambda i: (i, 0)),
)


@functools.partial(
    pl.kernel,
    out_type=(),
    mesh=_mesh,
    scratch_types=[
        pltpu.VMEM((_LCAP,), jnp.int32),
        pltpu.VMEM((_LCAP,), jnp.int32),
        pltpu.VMEM((16,), jnp.int32),
        pltpu.SemaphoreType.DMA,
    ],
    compiler_params=pltpu.CompilerParams(needs_layout_passes=False),
)
def _sc_apply(out_ref, rowl_hbm, wl_hbm, cnt_hbm, upd_hbm,
              rowlist_v, wlist_v, cnt_v, sd):
    g = lax.axis_index("s") * 2 + lax.axis_index("c")
    lane = lax.iota(jnp.int32, 16)

    pltpu.sync_copy(cnt_hbm.at[g], cnt_v)
    n = jnp.max(cnt_v[pl.ds(0, 16)])

    @pl.when(n > 0)
    def _apply():
        pltpu.sync_copy(rowl_hbm.at[g], rowlist_v)
        pltpu.sync_copy(wl_hbm.at[g], wlist_v)
        nchunks = (n + _CHUNK - 1) // _CHUNK

        @pl.loop(0, nchunks)
        def _chunk(c):
            off = c * _CHUNK
            for t in range(_CHUNK // 16):
                rv = rowlist_v[pl.ds(off + t * 16, 16)]
                wv = wlist_v[pl.ds(off + t * 16, 16)]

                @pl.loop(0, 16)
                def _fire(j2, rv=rv, wv=wv):
                    r = jnp.max(jnp.where(lane == j2, rv, -1))
                    w = jnp.max(jnp.where(lane == j2, wv, -1))
                    pltpu.make_async_copy(
                        upd_hbm.at[pl.ds(w, 1)],
                        out_ref.at[pl.ds(r, 1)],
                        sd,
                    ).start()

            @pl.loop(0, _CHUNK)
            def _drain(j):
                pltpu.make_async_copy(
                    upd_hbm.at[pl.ds(0, 1)],
                    out_ref.at[pl.ds(0, 1)],
                    sd,
                ).wait()


def kernel(data, indices, updates):
    idx = indices.reshape(_B)
    rowl, wl, cnt = _sc_prep(idx)
    out0 = _tc_copy(data)
    r = jax.new_ref(out0)
    _sc_apply(r, rowl, wl, cnt, updates)
    return r[...]
